# Initial kernel scaffold; baseline (speedup 1.0000x reference)
import jax
import jax.numpy as jnp
from jax.experimental import pallas as pl


def kernel(hidden_states, W_gate, W1, b1, W2, b2):
    raise NotImplementedError("write your pallas kernel here")


# speedup vs baseline: 5.0303x; 5.0303x over previous
import functools

import jax
import jax.numpy as jnp
from jax.experimental import pallas as pl
from jax.experimental.pallas import tpu as pltpu

_B, _S, _D, _E, _DFF, _TOPK = 1, 2048, 1024, 8, 2048, 2
_TS = 512
_NS = _S // _TS


def _route_kernel(x_ref, wg_ref, logits_ref, wts_ref, idx_ref):
    x = x_ref[...]
    r = jnp.dot(x, wg_ref[...], preferred_element_type=jnp.float32)
    logits = jnp.mean(r, axis=0, keepdims=True)
    logits_ref[...] = logits
    m = jnp.max(logits)
    ex = jnp.exp(logits - m)
    p = ex / jnp.sum(ex)
    i1 = jnp.argmax(p)
    w1 = jnp.max(p)
    iota = jax.lax.broadcasted_iota(jnp.int32, (1, _E), 1)
    p2 = jnp.where(iota == i1, -jnp.inf, p)
    i2 = jnp.argmax(p2)
    w2 = jnp.max(p2)
    wts_ref[...] = jnp.concatenate(
        [w1.reshape(1, 1), w2.reshape(1, 1)], axis=1)
    idx_ref[...] = jnp.concatenate(
        [i1.astype(jnp.int32).reshape(1, 1), i2.astype(jnp.int32).reshape(1, 1)],
        axis=1)


def _ffn_kernel(idx_sm, wts_sm, x_ref, w1_ref, b1_ref, w2_ref, b2_ref, out_ref):
    k = pl.program_id(0)
    si = pl.program_id(1)
    x = x_ref[...].astype(jnp.bfloat16)
    w1 = w1_ref[0].astype(jnp.bfloat16)
    h = jnp.dot(x, w1, preferred_element_type=jnp.float32)
    h = jax.nn.gelu(h + b1_ref[0])
    o = jnp.dot(h.astype(jnp.bfloat16), w2_ref[0].astype(jnp.bfloat16),
                preferred_element_type=jnp.float32)
    o = o + b2_ref[0]
    contrib = wts_sm[k] * o

    @pl.when(k == 0)
    def _():
        out_ref[pl.ds(si * _TS, _TS), :] = contrib

    @pl.when(k > 0)
    def _():
        out_ref[pl.ds(si * _TS, _TS), :] = (
            out_ref[pl.ds(si * _TS, _TS), :] + contrib)


@jax.jit
def kernel(hidden_states, W_gate, W1, b1, W2, b2):
    x2 = hidden_states.reshape(_S, _D)

    logits, wts, idx = pl.pallas_call(
        _route_kernel,
        out_shape=(
            jax.ShapeDtypeStruct((1, _E), jnp.float32),
            jax.ShapeDtypeStruct((1, _TOPK), jnp.float32),
            jax.ShapeDtypeStruct((1, _TOPK), jnp.int32),
        ),
    )(x2, W_gate)

    grid_spec = pltpu.PrefetchScalarGridSpec(
        num_scalar_prefetch=2,
        grid=(_TOPK, _NS),
        in_specs=[
            pl.BlockSpec((_TS, _D), lambda k, si, idx_s, wts_s: (si, 0)),
            pl.BlockSpec((1, _D, _DFF), lambda k, si, idx_s, wts_s: (idx_s[k], 0, 0)),
            pl.BlockSpec((1, 1, _DFF), lambda k, si, idx_s, wts_s: (idx_s[k], 0, 0)),
            pl.BlockSpec((1, _DFF, _D), lambda k, si, idx_s, wts_s: (idx_s[k], 0, 0)),
            pl.BlockSpec((1, 1, _D), lambda k, si, idx_s, wts_s: (idx_s[k], 0, 0)),
        ],
        out_specs=pl.BlockSpec((_S, _D), lambda k, si, idx_s, wts_s: (0, 0)),
    )
    out = pl.pallas_call(
        _ffn_kernel,
        grid_spec=grid_spec,
        out_shape=jax.ShapeDtypeStruct((_S, _D), jnp.float32),
        compiler_params=pltpu.CompilerParams(
            dimension_semantics=("arbitrary", "arbitrary")),
    )(idx.reshape(_TOPK), wts.reshape(_TOPK), x2, W1,
      b1.reshape(_E, 1, _DFF), W2, b2.reshape(_E, 1, _D))

    return (out.reshape(_B, _S, _D), logits)
